# Initial kernel scaffold; baseline (speedup 1.0000x reference)
import jax
import jax.numpy as jnp
from jax.experimental import pallas as pl


def kernel(x, w1, w2):
    raise NotImplementedError("write your pallas kernel here")


# speedup vs baseline: 1.0835x; 1.0835x over previous
import functools

import jax
import jax.numpy as jnp
from jax.experimental import pallas as pl
from jax.experimental.pallas import tpu as pltpu


def _se_kernel(x_ref, w1_ref, w2_ref, o_ref, *, inv_hw):
    xt = x_ref[...]
    s = jnp.sum(xt, axis=1, keepdims=True)
    y1 = jnp.dot(w1_ref[...], s * inv_hw, preferred_element_type=jnp.float32)
    y1 = jnp.maximum(y1, 0.0)
    y2 = jnp.dot(w2_ref[...], y1, preferred_element_type=jnp.float32)
    gate = 1.0 / (1.0 + jnp.exp(-y2))
    o_ref[...] = xt * gate


def kernel(x, w1, w2):
    B, C, H, W = x.shape
    Cr = w1.shape[0]
    HW = H * W
    xr = x.reshape(B, C, HW)

    out = pl.pallas_call(
        functools.partial(_se_kernel, inv_hw=1.0 / float(HW)),
        out_shape=jax.ShapeDtypeStruct((B, C, HW), x.dtype),
        grid=(B,),
        in_specs=[
            pl.BlockSpec((None, C, HW), lambda b: (b, 0, 0)),
            pl.BlockSpec((Cr, C), lambda b: (0, 0)),
            pl.BlockSpec((C, Cr), lambda b: (0, 0)),
        ],
        out_specs=pl.BlockSpec((None, C, HW), lambda b: (b, 0, 0)),
        compiler_params=pltpu.CompilerParams(
            dimension_semantics=("parallel",),
            vmem_limit_bytes=64 << 20),
    )(xr, w1, w2)
    return out.reshape(B, C, H, W)
